# Initial kernel scaffold; baseline (speedup 1.0000x reference)
import functools

import jax
import jax.numpy as jnp
from jax import lax
from jax.experimental import pallas as pl
from jax.experimental.pallas import tpu as pltpu
from jax.experimental.pallas import tpu_sc as plsc

_N = 10000
_E = 320000
_H = 128
_RBF = 50
_TYPES = 100
_CUT = 10.0

_NC, _NS = 2, 16
_NW = _NC * _NS
_G = 128
_NGROUP = _E // _G
_GPW = _NGROUP // _NW
_EXTRA = _NGROUP - _GPW * _NW
_RPS = _N // _NS

_RB = 2000
_EC = 2560


def _embed_body(t_ref, emb_ref, w10_ref, x_ref, h_ref):
    t = t_ref[...]
    oh = (t == lax.broadcasted_iota(jnp.int32, (_RB, _TYPES), 1)).astype(jnp.float32)
    x = oh @ emb_ref[...]
    x_ref[...] = x
    h_ref[...] = x @ w10_ref[...]


def _embed(atom_types, emb, w10):
    return pl.pallas_call(
        _embed_body,
        grid=(_N // _RB,),
        in_specs=[
            pl.BlockSpec((_RB, 1), lambda i: (i, 0)),
            pl.BlockSpec((_TYPES, _H), lambda i: (0, 0)),
            pl.BlockSpec((_H, _H), lambda i: (0, 0)),
        ],
        out_specs=[pl.BlockSpec((_RB, _H), lambda i: (i, 0))] * 2,
        out_shape=[jax.ShapeDtypeStruct((_N, _H), jnp.float32)] * 2,
    )(atom_types.reshape(_N, 1), emb, w10)


def _filter_body(w_ref, wf1_ref, bf1_ref, wf2_ref, bf2_ref, o0_ref, o1_ref, o2_ref):
    w = w_ref[...]
    step = _CUT / (_RBF - 1)
    coeff = -0.5 / (step * step)
    offs = lax.broadcasted_iota(jnp.float32, (_EC, _RBF), 1) * step
    d = w - offs
    rbf = jnp.exp(coeff * d * d)
    env = 0.5 * (jnp.cos(jnp.pi / _CUT * w) + 1.0) * (w < _CUT).astype(jnp.float32)
    for o_ref, i in ((o0_ref, 0), (o1_ref, 1), (o2_ref, 2)):
        t = jnp.tanh(rbf @ wf1_ref[...][i] + bf1_ref[...][i])
        o_ref[...] = (t @ wf2_ref[...][i] + bf2_ref[...][i]) * env


def _filters(edge_weight, wf1, bf1, wf2, bf2):
    return pl.pallas_call(
        _filter_body,
        grid=(_E // _EC,),
        in_specs=[
            pl.BlockSpec((_EC, 1), lambda i: (i, 0)),
            pl.BlockSpec((3, _RBF, _H), lambda i: (0, 0, 0)),
            pl.BlockSpec((3, _H), lambda i: (0, 0)),
            pl.BlockSpec((3, _H, _H), lambda i: (0, 0, 0)),
            pl.BlockSpec((3, _H), lambda i: (0, 0)),
        ],
        out_specs=[pl.BlockSpec((_EC, _H), lambda i: (i, 0))] * 3,
        out_shape=[jax.ShapeDtypeStruct((_E, _H), jnp.float32)] * 3,
    )(edge_weight.reshape(_E, 1), wf1, bf1, wf2, bf2)


def _update_body(p_ref, x_ref, w2_ref, b2_ref, wl_ref, bl_ref, w1n_ref, xo_ref, ho_ref):
    agg = p_ref[...][0] + p_ref[...][1]
    h2 = jnp.tanh(agg @ w2_ref[...] + b2_ref[...]) @ wl_ref[...] + bl_ref[...]
    xn = x_ref[...] + h2
    xo_ref[...] = xn
    ho_ref[...] = xn @ w1n_ref[...]


def _update(p, x, w2, b2, wl, bl, w1n):
    return pl.pallas_call(
        _update_body,
        grid=(_N // _RB,),
        in_specs=[
            pl.BlockSpec((_NC, _RB, _H), lambda i: (0, i, 0)),
            pl.BlockSpec((_RB, _H), lambda i: (i, 0)),
            pl.BlockSpec((_H, _H), lambda i: (0, 0)),
            pl.BlockSpec((1, _H), lambda i: (0, 0)),
            pl.BlockSpec((_H, _H), lambda i: (0, 0)),
            pl.BlockSpec((1, _H), lambda i: (0, 0)),
            pl.BlockSpec((_H, _H), lambda i: (0, 0)),
        ],
        out_specs=[pl.BlockSpec((_RB, _H), lambda i: (i, 0))] * 2,
        out_shape=[jax.ShapeDtypeStruct((_N, _H), jnp.float32)] * 2,
    )(p, x, w2, b2.reshape(1, _H), wl, bl.reshape(1, _H), w1n)


def _final_body(p_ref, x_ref, w2_ref, b2_ref, wl_ref, bl_ref,
                wo1_ref, bo1_ref, wo2_ref, bo2_ref, o_ref):
    agg = p_ref[...][0] + p_ref[...][1]
    h2 = jnp.tanh(agg @ w2_ref[...] + b2_ref[...]) @ wl_ref[...] + bl_ref[...]
    xn = x_ref[...] + h2
    o_ref[...] = jnp.tanh(xn @ wo1_ref[...] + bo1_ref[...]) @ wo2_ref[...] + bo2_ref[...]


def _final(p, x, w2, b2, wl, bl, wo1, bo1, wo2, bo2):
    return pl.pallas_call(
        _final_body,
        grid=(_N // _RB,),
        in_specs=[
            pl.BlockSpec((_NC, _RB, _H), lambda i: (0, i, 0)),
            pl.BlockSpec((_RB, _H), lambda i: (i, 0)),
            pl.BlockSpec((_H, _H), lambda i: (0, 0)),
            pl.BlockSpec((1, _H), lambda i: (0, 0)),
            pl.BlockSpec((_H, _H), lambda i: (0, 0)),
            pl.BlockSpec((1, _H), lambda i: (0, 0)),
            pl.BlockSpec((_H, 64), lambda i: (0, 0)),
            pl.BlockSpec((1, 64), lambda i: (0, 0)),
            pl.BlockSpec((64, 1), lambda i: (0, 0)),
            pl.BlockSpec((1, 1), lambda i: (0, 0)),
        ],
        out_specs=pl.BlockSpec((_RB, 1), lambda i: (i, 0)),
        out_shape=jax.ShapeDtypeStruct((_N, 1), jnp.float32),
    )(p, x, w2, b2.reshape(1, _H), wl, bl.reshape(1, _H),
      wo1, bo1.reshape(1, 64), wo2, bo2.reshape(1, 1))


def _sc_body(h_hbm, wf_hbm, src_hbm, dst_hbm, z_hbm, out_hbm,
             acc, src_v, dst_v, wf_v, rows_v, obuf, sem):
    c = lax.axis_index("c")
    s = lax.axis_index("s")
    w = s * _NC + c

    pltpu.sync_copy(z_hbm, obuf)
    for j in range(_RPS // 125):
        pltpu.sync_copy(obuf, acc.at[pl.ds(s * _RPS + j * 125, 125)])
    plsc.subcore_barrier()

    nmy = _GPW + jnp.where(w < _EXTRA, 1, 0)
    start = w * _GPW + jnp.minimum(w, _EXTRA)

    def group(gi, carry):
        g = start + gi
        pltpu.sync_copy(src_hbm.at[g], src_v)
        pltpu.sync_copy(dst_hbm.at[g], dst_v)
        cp_f = pltpu.async_copy(wf_hbm.at[pl.ds(g * _G, _G)], wf_v, sem)
        cp_g = pltpu.async_copy(h_hbm.at[src_v], rows_v, sem)
        cp_f.wait()
        cp_g.wait()

        def mulrow(r, carry2):
            for cc in range(_H // 16):
                sl = pl.ds(cc * 16, 16)
                rows_v[r, sl] = rows_v[r, sl] * wf_v[r, sl]
            return carry2

        lax.fori_loop(0, _G, mulrow, 0)
        pltpu.sync_copy(rows_v, acc.at[dst_v], add=True)
        return carry

    lax.fori_loop(0, nmy, group, 0)
    plsc.subcore_barrier()

    for j in range(_RPS // 125):
        r0 = s * _RPS + j * 125
        pltpu.sync_copy(acc.at[pl.ds(r0, 125)], obuf)
        pltpu.sync_copy(obuf, out_hbm.at[c, pl.ds(r0, 125)])


@functools.partial(
    pl.kernel,
    out_type=jax.ShapeDtypeStruct((_NC, _N, _H), jnp.float32),
    mesh=plsc.VectorSubcoreMesh(
        core_axis_name="c", subcore_axis_name="s",
        num_cores=_NC, num_subcores=_NS),
    scratch_types=[
        pltpu.VMEM_SHARED((_N, _H), jnp.float32),
        pltpu.VMEM((_G,), jnp.int32),
        pltpu.VMEM((_G,), jnp.int32),
        pltpu.VMEM((_G, _H), jnp.float32),
        pltpu.VMEM((_G, _H), jnp.float32),
        pltpu.VMEM((125, _H), jnp.float32),
        pltpu.SemaphoreType.DMA,
    ],
)
def _sc_scatter(h_hbm, wf_hbm, src_hbm, dst_hbm, z_hbm, out_hbm,
                acc, src_v, dst_v, wf_v, rows_v, obuf, sem):
    _sc_body(h_hbm, wf_hbm, src_hbm, dst_hbm, z_hbm, out_hbm,
             acc, src_v, dst_v, wf_v, rows_v, obuf, sem)


def kernel(atom_types, edge_index, edge_weight, emb, W1, Wf1, bf1, Wf2, bf2,
           W2, b2, Wl, bl, Wo1, bo1, Wo2, bo2):
    src2d = edge_index[0].astype(jnp.int32).reshape(_NGROUP, _G)
    dst2d = edge_index[1].astype(jnp.int32).reshape(_NGROUP, _G)
    zeros = jnp.zeros((125, _H), jnp.float32)

    x, h = _embed(atom_types.astype(jnp.int32), emb, W1[0])
    wfs = _filters(edge_weight, Wf1, bf1, Wf2, bf2)

    out = None
    for i in range(3):
        p = _sc_scatter(h, wfs[i], src2d, dst2d, zeros)
        if i < 2:
            x, h = _update(p, x, W2[i], b2[i], Wl[i], bl[i], W1[i + 1])
        else:
            out = _final(p, x, W2[i], b2[i], Wl[i], bl[i], Wo1, bo1, Wo2, bo2)
    return out


# speedup vs baseline: 2.7531x; 2.7531x over previous
import functools

import jax
import jax.numpy as jnp
from jax import lax
from jax.experimental import pallas as pl
from jax.experimental.pallas import tpu as pltpu
from jax.experimental.pallas import tpu_sc as plsc

_N = 10000
_E = 320000
_H = 128
_RBF = 50
_TYPES = 100
_CUT = 10.0

_NC, _NS = 2, 16
_NW = _NC * _NS
_G = 128
_SG = 8
_NSG = 313
_E_PAD = _NSG * _SG * _G
_SGPW = _NSG // _NW
_EXTRA = _NSG - _SGPW * _NW
_NP = 10240
_RPS = _NP // _NS
_DC = 64

_RB = 2000
_EC = 2504


def _embed_body(t_ref, emb_ref, w10_ref, x_ref, h_ref):
    t = t_ref[...]
    oh = (t == lax.broadcasted_iota(jnp.int32, (_RB, _TYPES), 1)).astype(jnp.float32)
    x = oh @ emb_ref[...]
    x_ref[...] = x
    h_ref[...] = x @ w10_ref[...]


def _embed(atom_types, emb, w10):
    return pl.pallas_call(
        _embed_body,
        grid=(_N // _RB,),
        in_specs=[
            pl.BlockSpec((_RB, 1), lambda i: (i, 0)),
            pl.BlockSpec((_TYPES, _H), lambda i: (0, 0)),
            pl.BlockSpec((_H, _H), lambda i: (0, 0)),
        ],
        out_specs=[pl.BlockSpec((_RB, _H), lambda i: (i, 0))] * 2,
        out_shape=[jax.ShapeDtypeStruct((_N, _H), jnp.float32)] * 2,
    )(atom_types.reshape(_N, 1), emb, w10)


def _filter_body(w_ref, wf1_ref, bf1_ref, wf2_ref, bf2_ref, o0_ref, o1_ref, o2_ref):
    w = w_ref[...]
    step = _CUT / (_RBF - 1)
    coeff = -0.5 / (step * step)
    offs = lax.broadcasted_iota(jnp.int32, (_EC, _RBF), 1).astype(jnp.float32) * step
    d = w - offs
    rbf = jnp.exp(coeff * d * d)
    env = 0.5 * (jnp.cos(jnp.pi / _CUT * w) + 1.0) * (w < _CUT).astype(jnp.float32)
    for o_ref, i in ((o0_ref, 0), (o1_ref, 1), (o2_ref, 2)):
        t = jnp.tanh(rbf @ wf1_ref[...][i] + bf1_ref[...][i])
        o_ref[...] = (t @ wf2_ref[...][i] + bf2_ref[...][i]) * env


def _filters(edge_weight_pad, wf1, bf1, wf2, bf2):
    return pl.pallas_call(
        _filter_body,
        grid=(_E_PAD // _EC,),
        in_specs=[
            pl.BlockSpec((_EC, 1), lambda i: (i, 0)),
            pl.BlockSpec((3, _RBF, _H), lambda i: (0, 0, 0)),
            pl.BlockSpec((3, _H), lambda i: (0, 0)),
            pl.BlockSpec((3, _H, _H), lambda i: (0, 0, 0)),
            pl.BlockSpec((3, _H), lambda i: (0, 0)),
        ],
        out_specs=[pl.BlockSpec((_EC, _H), lambda i: (i, 0))] * 3,
        out_shape=[jax.ShapeDtypeStruct((_E_PAD, _H), jnp.float32)] * 3,
    )(edge_weight_pad.reshape(_E_PAD, 1), wf1, bf1, wf2, bf2)


def _update_body(p_ref, x_ref, w2_ref, b2_ref, wl_ref, bl_ref, w1n_ref, xo_ref, ho_ref):
    agg = p_ref[...][0] + p_ref[...][1]
    h2 = jnp.tanh(agg @ w2_ref[...] + b2_ref[...]) @ wl_ref[...] + bl_ref[...]
    xn = x_ref[...] + h2
    xo_ref[...] = xn
    ho_ref[...] = xn @ w1n_ref[...]


def _update(p, x, w2, b2, wl, bl, w1n):
    return pl.pallas_call(
        _update_body,
        grid=(_N // _RB,),
        in_specs=[
            pl.BlockSpec((_NC, _RB, _H), lambda i: (0, i, 0)),
            pl.BlockSpec((_RB, _H), lambda i: (i, 0)),
            pl.BlockSpec((_H, _H), lambda i: (0, 0)),
            pl.BlockSpec((1, _H), lambda i: (0, 0)),
            pl.BlockSpec((_H, _H), lambda i: (0, 0)),
            pl.BlockSpec((1, _H), lambda i: (0, 0)),
            pl.BlockSpec((_H, _H), lambda i: (0, 0)),
        ],
        out_specs=[pl.BlockSpec((_RB, _H), lambda i: (i, 0))] * 2,
        out_shape=[jax.ShapeDtypeStruct((_N, _H), jnp.float32)] * 2,
    )(p, x, w2, b2.reshape(1, _H), wl, bl.reshape(1, _H), w1n)


def _final_body(p_ref, x_ref, w2_ref, b2_ref, wl_ref, bl_ref,
                wo1_ref, bo1_ref, wo2_ref, bo2_ref, o_ref):
    agg = p_ref[...][0] + p_ref[...][1]
    h2 = jnp.tanh(agg @ w2_ref[...] + b2_ref[...]) @ wl_ref[...] + bl_ref[...]
    xn = x_ref[...] + h2
    o_ref[...] = jnp.tanh(xn @ wo1_ref[...] + bo1_ref[...]) @ wo2_ref[...] + bo2_ref[...]


def _final(p, x, w2, b2, wl, bl, wo1, bo1, wo2, bo2):
    return pl.pallas_call(
        _final_body,
        grid=(_N // _RB,),
        in_specs=[
            pl.BlockSpec((_NC, _RB, _H), lambda i: (0, i, 0)),
            pl.BlockSpec((_RB, _H), lambda i: (i, 0)),
            pl.BlockSpec((_H, _H), lambda i: (0, 0)),
            pl.BlockSpec((1, _H), lambda i: (0, 0)),
            pl.BlockSpec((_H, _H), lambda i: (0, 0)),
            pl.BlockSpec((1, _H), lambda i: (0, 0)),
            pl.BlockSpec((_H, 64), lambda i: (0, 0)),
            pl.BlockSpec((1, 64), lambda i: (0, 0)),
            pl.BlockSpec((64, 1), lambda i: (0, 0)),
            pl.BlockSpec((1, 1), lambda i: (0, 0)),
        ],
        out_specs=pl.BlockSpec((_RB, 1), lambda i: (i, 0)),
        out_shape=jax.ShapeDtypeStruct((_N, 1), jnp.float32),
    )(p, x, w2, b2.reshape(1, _H), wl, bl.reshape(1, _H),
      wo1, bo1.reshape(1, 64), wo2, bo2.reshape(1, 1))


def _sc_body(h_hbm, wf_hbm, src_hbm, dst_hbm, z_hbm, out_hbm,
             acc, src_v, dst_v, wf_v, rows_v, obuf, sem):
    c = lax.axis_index("c")
    s = lax.axis_index("s")
    w = s * _NC + c

    pltpu.sync_copy(z_hbm, obuf)
    for j in range(_RPS // _DC):
        pltpu.sync_copy(obuf, acc.at[pl.ds(s * _RPS + j * _DC, _DC)])
    plsc.subcore_barrier()

    nmy = _SGPW + jnp.where(w < _EXTRA, 1, 0)
    start = w * _SGPW + jnp.minimum(w, _EXTRA)

    def supergroup(si, carry):
        sg = start + si
        pltpu.sync_copy(src_hbm.at[sg], src_v)
        pltpu.sync_copy(dst_hbm.at[sg], dst_v)
        for j in range(_SG):
            g = sg * _SG + j
            cp_f = pltpu.async_copy(wf_hbm.at[pl.ds(g * _G, _G)], wf_v, sem)
            cp_g = pltpu.async_copy(h_hbm.at[src_v.at[j]], rows_v, sem)
            cp_f.wait()
            cp_g.wait()

            def mulrow(r, carry2):
                for cc in range(_H // 16):
                    sl = pl.ds(cc * 16, 16)
                    rows_v[r, sl] = rows_v[r, sl] * wf_v[r, sl]
                return carry2

            lax.fori_loop(0, _G, mulrow, 0)
            pltpu.sync_copy(rows_v, acc.at[dst_v.at[j]], add=True)
        return carry

    lax.fori_loop(0, nmy, supergroup, 0)
    plsc.subcore_barrier()

    for j in range(_RPS // _DC):
        r0 = s * _RPS + j * _DC
        pltpu.sync_copy(acc.at[pl.ds(r0, _DC)], obuf)
        pltpu.sync_copy(obuf, out_hbm.at[c, pl.ds(r0, _DC)])


@functools.cache
def _make_sc_scatter():
    return pl.kernel(
        _sc_body,
        out_type=jax.ShapeDtypeStruct((_NC, _NP, _H), jnp.float32),
        mesh=plsc.VectorSubcoreMesh(
            core_axis_name="c", subcore_axis_name="s",
            num_cores=_NC, num_subcores=_NS),
        scratch_types=[
            pltpu.VMEM_SHARED((_NP, _H), jnp.float32),
            pltpu.VMEM((_SG, _G), jnp.int32),
            pltpu.VMEM((_SG, _G), jnp.int32),
            pltpu.VMEM((_G, _H), jnp.float32),
            pltpu.VMEM((_G, _H), jnp.float32),
            pltpu.VMEM((_DC, _H), jnp.float32),
            pltpu.SemaphoreType.DMA,
        ],
    )


def _sc_scatter(h, wf, src3d, dst3d, zeros):
    return _make_sc_scatter()(h, wf, src3d, dst3d, zeros)


def kernel(atom_types, edge_index, edge_weight, emb, W1, Wf1, bf1, Wf2, bf2,
           W2, b2, Wl, bl, Wo1, bo1, Wo2, bo2):
    npad = _E_PAD - _E
    ei = edge_index.astype(jnp.int32)
    src3d = jnp.concatenate([ei[0], jnp.zeros((npad,), jnp.int32)]).reshape(_NSG, _SG, _G)
    dst3d = jnp.concatenate([ei[1], jnp.zeros((npad,), jnp.int32)]).reshape(_NSG, _SG, _G)
    ew_pad = jnp.concatenate(
        [edge_weight, jnp.full((npad,), 2.0 * _CUT, jnp.float32)])
    zeros = jnp.zeros((_DC, _H), jnp.float32)

    x, h = _embed(atom_types.astype(jnp.int32), emb, W1[0])
    wfs = _filters(ew_pad, Wf1, bf1, Wf2, bf2)

    out = None
    for i in range(3):
        p = _sc_scatter(h, wfs[i], src3d, dst3d, zeros)
        if i < 2:
            x, h = _update(p, x, W2[i], b2[i], Wl[i], bl[i], W1[i + 1])
        else:
            out = _final(p, x, W2[i], b2[i], Wl[i], bl[i], Wo1, bo1, Wo2, bo2)
    return out
